# baseline (speedup 1.0000x reference)
import jax
import jax.numpy as jnp
import numpy as np
from jax.experimental import pallas as pl
from jax.experimental.pallas import tpu as pltpu

T = 1000
TPAD = 1024
B = 128
R = 1176
C = 128
L = R * C
BB = 2

_K1 = np.uint32(42)
_K2 = np.uint32(0x1BD11BDA ^ 42)
_LO = np.float32(np.nextafter(np.float32(-1.0), np.float32(0.0)))

_CC = (np.float32(2.122917214274262), np.float32(-0.34995386083658697),
       np.float32(-0.004681780622241893), np.float32(0.0021330589779123277))
_CT = (np.float32(4.005365305566973), np.float32(1.4192557312029732),
       np.float32(0.032923790098936645))


def _sched_body(ts_ref, beta_ref, sa_ref, sb_ref):
    la = jnp.log1p(-beta_ref[0, :])
    i = jax.lax.broadcasted_iota(jnp.int32, (B, TPAD), 1)
    mask = i <= ts_ref[...]
    s = jnp.sum(jnp.where(mask, la[None, :], 0.0), axis=1, keepdims=True)
    cp = jnp.exp(s)
    sa_ref[...] = jnp.sqrt(cp)
    sb_ref[...] = jnp.sqrt(1.0 - cp)


def _rotl(v, r):
    return (v << np.uint32(r)) | (v >> np.uint32(32 - r))


def _main_body(sa_ref, sb_ref, x_ref, out_ref, noise_ref):
    pid = pl.program_id(0)
    ir = jax.lax.broadcasted_iota(jnp.uint32, (R, C), 0)
    ic = jax.lax.broadcasted_iota(jnp.uint32, (R, C), 1)
    jbase = ir * np.uint32(C) + ic
    for bi in range(BB):
        b = pid * BB + bi
        base = (b * L + 42).astype(jnp.uint32)
        x1 = jbase + base
        x0 = x1
        x1 = _rotl(x1, 13) ^ x0
        for r in (15, 26, 6):
            x0 = x0 + x1
            x1 = _rotl(x1, r) ^ x0
        x0 = x0 + _K1
        x1 = x1 + np.uint32(_K2 + 1)
        for r in (17, 29, 16, 24):
            x0 = x0 + x1
            x1 = _rotl(x1, r) ^ x0
        x0 = x0 + _K2
        x1 = x1 + np.uint32(2)
        for r in (13, 15, 26, 6):
            x0 = x0 + x1
            x1 = _rotl(x1, r) ^ x0
        x1 = x1 + np.uint32(_K1 + 3)
        for r in (17, 29, 16, 24):
            x0 = x0 + x1
            x1 = _rotl(x1, r) ^ x0
        x0 = x0 + _K1
        x1 = x1 + np.uint32(_K2 + 4)
        for r in (13, 15, 26, 6):
            x0 = x0 + x1
            x1 = _rotl(x1, r) ^ x0
        x0 = x0 + _K2
        x1 = x1 + np.uint32(5)
        bits = x0 ^ x1

        g = jax.lax.bitcast_convert_type((bits >> np.uint32(9))
                                         | np.uint32(0x40000000), jnp.float32)
        u = jnp.maximum(g - np.float32(3.0), _LO)
        y = jnp.log(1.0 - u * u)
        q = y + np.float32(2.5)
        pc = ((_CC[3] * q + _CC[2]) * q + _CC[1]) * q + _CC[0]
        st = jnp.sqrt(-y) - np.float32(3.0)
        pt = (_CT[2] * st + _CT[1]) * st + _CT[0]
        p = jnp.where(y > np.float32(-5.0), pc, pt)
        z = u * p

        noise_ref[bi] = z
        sa = sa_ref[b, 0]
        sb = sb_ref[b, 0]
        out_ref[bi] = sa * x_ref[bi] + sb * z


@jax.jit
def kernel(x, timestep, beta_schedule):
    ts = timestep.reshape(B, 1)
    beta = jnp.pad(beta_schedule, (0, TPAD - T)).reshape(1, TPAD)
    sa, sb = pl.pallas_call(
        _sched_body,
        in_specs=[
            pl.BlockSpec((B, 1), lambda: (0, 0)),
            pl.BlockSpec((1, TPAD), lambda: (0, 0)),
        ],
        out_specs=[
            pl.BlockSpec((B, 1), lambda: (0, 0)),
            pl.BlockSpec((B, 1), lambda: (0, 0)),
        ],
        out_shape=[
            jax.ShapeDtypeStruct((B, 1), jnp.float32),
            jax.ShapeDtypeStruct((B, 1), jnp.float32),
        ],
    )(ts, beta)

    x3 = x.reshape(B, R, C)
    out, noise = pl.pallas_call(
        _main_body,
        grid=(B // BB,),
        in_specs=[
            pl.BlockSpec(memory_space=pltpu.SMEM),
            pl.BlockSpec(memory_space=pltpu.SMEM),
            pl.BlockSpec((BB, R, C), lambda i: (i, 0, 0)),
        ],
        out_specs=[
            pl.BlockSpec((BB, R, C), lambda i: (i, 0, 0)),
            pl.BlockSpec((BB, R, C), lambda i: (i, 0, 0)),
        ],
        out_shape=[
            jax.ShapeDtypeStruct((B, R, C), x.dtype),
            jax.ShapeDtypeStruct((B, R, C), x.dtype),
        ],
    )(sa, sb, x3)
    return out.reshape(x.shape), noise.reshape(x.shape)


# speedup vs baseline: 1.0319x; 1.0319x over previous
import jax
import jax.numpy as jnp
import numpy as np
from jax.experimental import pallas as pl
from jax.experimental.pallas import tpu as pltpu

T = 1000
TPAD = 1024
B = 128
R = 1176
C = 128
L = R * C
BB = 2

_K1 = np.uint32(42)
_K2 = np.uint32(0x1BD11BDA ^ 42)

_PC = (np.float32(1.253345163694818), np.float32(-0.3274664588756707),
       np.float32(0.018076108232483987), np.float32(0.0048386815075306365),
       np.float32(0.0003785217596220941), np.float32(1.0414633647648497e-05))
_UMAX = np.float32(0.99999994)


def _sched_body(ts_ref, beta_ref, sa_ref, sb_ref):
    la = jnp.log1p(-beta_ref[0, :])
    i = jax.lax.broadcasted_iota(jnp.int32, (B, TPAD), 1)
    mask = i <= ts_ref[...]
    s = jnp.sum(jnp.where(mask, la[None, :], 0.0), axis=1, keepdims=True)
    cp = jnp.exp(s)
    sa_ref[...] = jnp.sqrt(cp)
    sb_ref[...] = jnp.sqrt(1.0 - cp)


def _rotl(v, r):
    return (v << np.uint32(r)) | (v >> np.uint32(32 - r))


def _main_body(sa_ref, sb_ref, x_ref, out_ref, noise_ref):
    pid = pl.program_id(0)
    ir = jax.lax.broadcasted_iota(jnp.uint32, (R, C), 0)
    ic = jax.lax.broadcasted_iota(jnp.uint32, (R, C), 1)
    jbase = ir * np.uint32(C) + ic
    for bi in range(BB):
        b = pid * BB + bi
        base = (b * L + 42).astype(jnp.uint32)
        x1 = jbase + base
        x0 = x1
        x1 = _rotl(x1, 13) ^ x0
        for r in (15, 26, 6):
            x0 = x0 + x1
            x1 = _rotl(x1, r) ^ x0
        x0 = x0 + _K1
        x1 = x1 + np.uint32(_K2 + 1)
        for r in (17, 29, 16, 24):
            x0 = x0 + x1
            x1 = _rotl(x1, r) ^ x0
        x0 = x0 + _K2
        x1 = x1 + np.uint32(2)
        for r in (13, 15, 26, 6):
            x0 = x0 + x1
            x1 = _rotl(x1, r) ^ x0
        x1 = x1 + np.uint32(_K1 + 3)
        for r in (17, 29, 16, 24):
            x0 = x0 + x1
            x1 = _rotl(x1, r) ^ x0
        x0 = x0 + _K1
        x1 = x1 + np.uint32(_K2 + 4)
        for r in (13, 15, 26, 6):
            x0 = x0 + x1
            x1 = _rotl(x1, r) ^ x0
        x0 = x0 + _K2
        x1 = x1 + np.uint32(5)
        bits = x0 ^ x1

        uf = bits.astype(jnp.float32)
        u = uf * np.float32(2.0**-31) - np.float32(1.0)
        u = jnp.minimum(jnp.maximum(u, -_UMAX), _UMAX)
        y = jnp.log(1.0 - u * u)
        p = ((((_PC[5] * y + _PC[4]) * y + _PC[3]) * y + _PC[2]) * y
             + _PC[1]) * y + _PC[0]
        z = u * p

        noise_ref[bi] = z
        sa = sa_ref[b, 0]
        sb = sb_ref[b, 0]
        out_ref[bi] = sa * x_ref[bi] + sb * z


@jax.jit
def kernel(x, timestep, beta_schedule):
    ts = timestep.reshape(B, 1)
    beta = jnp.pad(beta_schedule, (0, TPAD - T)).reshape(1, TPAD)
    sa, sb = pl.pallas_call(
        _sched_body,
        in_specs=[
            pl.BlockSpec((B, 1), lambda: (0, 0)),
            pl.BlockSpec((1, TPAD), lambda: (0, 0)),
        ],
        out_specs=[
            pl.BlockSpec((B, 1), lambda: (0, 0)),
            pl.BlockSpec((B, 1), lambda: (0, 0)),
        ],
        out_shape=[
            jax.ShapeDtypeStruct((B, 1), jnp.float32),
            jax.ShapeDtypeStruct((B, 1), jnp.float32),
        ],
    )(ts, beta)

    x3 = x.reshape(B, R, C)
    out, noise = pl.pallas_call(
        _main_body,
        grid=(B // BB,),
        in_specs=[
            pl.BlockSpec(memory_space=pltpu.SMEM),
            pl.BlockSpec(memory_space=pltpu.SMEM),
            pl.BlockSpec((BB, R, C), lambda i: (i, 0, 0)),
        ],
        out_specs=[
            pl.BlockSpec((BB, R, C), lambda i: (i, 0, 0)),
            pl.BlockSpec((BB, R, C), lambda i: (i, 0, 0)),
        ],
        out_shape=[
            jax.ShapeDtypeStruct((B, R, C), x.dtype),
            jax.ShapeDtypeStruct((B, R, C), x.dtype),
        ],
    )(sa, sb, x3)
    return out.reshape(x.shape), noise.reshape(x.shape)


# speedup vs baseline: 1.0623x; 1.0295x over previous
import jax
import jax.numpy as jnp
import numpy as np
from jax.experimental import pallas as pl
from jax.experimental.pallas import tpu as pltpu

T = 1000
TPAD = 1024
B = 128
R = 1176
C = 128
L = R * C
BB = 2

_K1 = np.uint32(42)
_K2 = np.uint32(0x1BD11BDA ^ 42)

_PC = (np.float32(1.2533451), np.float32(-0.22698246),
       np.float32(0.008684721), np.float32(0.0016114003),
       np.float32(8.737611e-05), np.float32(1.6663695e-06))
_UMAX = np.float32(0.99999994)


def _sched_body(ts_ref, beta_ref, sa_ref, sb_ref):
    la = jnp.log1p(-beta_ref[0, :])
    i = jax.lax.broadcasted_iota(jnp.int32, (B, TPAD), 1)
    mask = i <= ts_ref[...]
    s = jnp.sum(jnp.where(mask, la[None, :], 0.0), axis=1, keepdims=True)
    cp = jnp.exp(s)
    sa_ref[...] = jnp.sqrt(cp)
    sb_ref[...] = jnp.sqrt(1.0 - cp)


def _rotl(v, r):
    return (v << np.uint32(r)) | (v >> np.uint32(32 - r))


def _main_body(sa_ref, sb_ref, jb_ref, x_ref, out_ref, noise_ref):
    pid = pl.program_id(0)
    for bi in range(BB):
        b = pid * BB + bi
        base = (b * L).astype(jnp.uint32)
        x1 = jb_ref[0] + base
        x0 = x1
        x1 = _rotl(x1, 13) ^ x0
        for r in (15, 26, 6):
            x0 = x0 + x1
            x1 = _rotl(x1, r) ^ x0
        x0 = x0 + _K1
        x1 = x1 + np.uint32(_K2 + 1)
        for r in (17, 29, 16, 24):
            x0 = x0 + x1
            x1 = _rotl(x1, r) ^ x0
        x0 = x0 + _K2
        x1 = x1 + np.uint32(2)
        for r in (13, 15, 26, 6):
            x0 = x0 + x1
            x1 = _rotl(x1, r) ^ x0
        x1 = x1 + np.uint32(_K1 + 3)
        for r in (17, 29, 16, 24):
            x0 = x0 + x1
            x1 = _rotl(x1, r) ^ x0
        x0 = x0 + _K1
        x1 = x1 + np.uint32(_K2 + 4)
        for r in (13, 15, 26, 6):
            x0 = x0 + x1
            x1 = _rotl(x1, r) ^ x0
        x0 = x0 + _K2
        x1 = x1 + np.uint32((5 + 0x80000000) & 0xFFFFFFFF)
        sbits = jax.lax.bitcast_convert_type(x0 ^ x1, jnp.int32)

        u = sbits.astype(jnp.float32) * np.float32(2.0 ** -31)
        u = jnp.minimum(jnp.maximum(u, -_UMAX), _UMAX)
        y = jnp.log2(1.0 - u * u)
        p = ((((_PC[5] * y + _PC[4]) * y + _PC[3]) * y + _PC[2]) * y
             + _PC[1]) * y + _PC[0]
        z = u * p

        noise_ref[bi] = z
        sa = sa_ref[b, 0]
        sb = sb_ref[b, 0]
        out_ref[bi] = sa * x_ref[bi] + sb * z


@jax.jit
def kernel(x, timestep, beta_schedule):
    ts = timestep.reshape(B, 1)
    beta = jnp.pad(beta_schedule, (0, TPAD - T)).reshape(1, TPAD)
    sa, sb = pl.pallas_call(
        _sched_body,
        in_specs=[
            pl.BlockSpec((B, 1), lambda: (0, 0)),
            pl.BlockSpec((1, TPAD), lambda: (0, 0)),
        ],
        out_specs=[
            pl.BlockSpec((B, 1), lambda: (0, 0)),
            pl.BlockSpec((B, 1), lambda: (0, 0)),
        ],
        out_shape=[
            jax.ShapeDtypeStruct((B, 1), jnp.float32),
            jax.ShapeDtypeStruct((B, 1), jnp.float32),
        ],
    )(ts, beta)

    jb = (jnp.arange(L, dtype=jnp.uint32) + jnp.uint32(42)).reshape(1, R, C)
    x3 = x.reshape(B, R, C)
    out, noise = pl.pallas_call(
        _main_body,
        grid=(B // BB,),
        in_specs=[
            pl.BlockSpec(memory_space=pltpu.SMEM),
            pl.BlockSpec(memory_space=pltpu.SMEM),
            pl.BlockSpec((1, R, C), lambda i: (0, 0, 0)),
            pl.BlockSpec((BB, R, C), lambda i: (i, 0, 0)),
        ],
        out_specs=[
            pl.BlockSpec((BB, R, C), lambda i: (i, 0, 0)),
            pl.BlockSpec((BB, R, C), lambda i: (i, 0, 0)),
        ],
        out_shape=[
            jax.ShapeDtypeStruct((B, R, C), x.dtype),
            jax.ShapeDtypeStruct((B, R, C), x.dtype),
        ],
    )(sa, sb, jb, x3)
    return out.reshape(x.shape), noise.reshape(x.shape)
